# baseline (speedup 1.0000x reference)
import functools

import jax
import jax.numpy as jnp
from jax import lax
from jax.experimental import pallas as pl
from jax.experimental.pallas import tpu as pltpu
from jax.experimental.pallas import tpu_sc as plsc

N_EMB = 8192
D = 256
N_TOK = 8192
BT = 256
BC = 512
GI = N_TOK // BT
GJ = N_EMB // BC
SCALE = 1.25 / (N_TOK * D)


def _vq_body(z_ref, en_ref, z2_ref, e2_ref, k_ref, loss_ref, acc_ref):
    i = pl.program_id(0)
    z = z_ref[...]
    z2 = z2_ref[...]
    lane = lax.broadcasted_iota(jnp.int32, (BT, 128), 1).astype(jnp.float32)
    rmin = None
    ridx = None
    for j in range(GJ):
        en = en_ref[pl.ds(j * BC, BC), :]
        dotj = lax.dot_general(z, en, (((1,), (1,)), ((), ())),
                               preferred_element_type=jnp.float32)
        scores = (z2 + dotj) + e2_ref[:, pl.ds(j * BC, BC)]
        m = scores[:, 0:128]
        c = jnp.zeros((BT, 128), jnp.float32)
        for t in range(1, BC // 128):
            st = scores[:, t * 128:(t + 1) * 128]
            lt = st < m
            m = jnp.where(lt, st, m)
            c = jnp.where(lt, jnp.float32(t), c)
        bmin = jnp.min(m, axis=1, keepdims=True)
        full = c * 128.0 + lane
        bidx = jnp.min(jnp.where(m == bmin, full, jnp.float32(BC)),
                       axis=1, keepdims=True).astype(jnp.int32) + j * BC
        if j == 0:
            rmin, ridx = bmin, bidx
        else:
            take = bmin < rmin
            rmin = jnp.where(take, bmin, rmin)
            ridx = jnp.where(take, bidx, ridx)
    k_ref[...] = ridx
    part = jnp.sum(rmin)
    tot = jnp.where(i == 0, part, acc_ref[0, 0] + part)
    acc_ref[0, 0] = tot

    @pl.when(i == GI - 1)
    def _():
        loss_ref[0, 0] = tot * SCALE


_vq = pl.pallas_call(
    _vq_body,
    grid=(GI,),
    in_specs=[
        pl.BlockSpec((BT, D), lambda i: (i, 0)),
        pl.BlockSpec((N_EMB, D), lambda i: (0, 0)),
        pl.BlockSpec((BT, 1), lambda i: (i, 0)),
        pl.BlockSpec((1, N_EMB), lambda i: (0, 0)),
    ],
    out_specs=[
        pl.BlockSpec((BT, 1), lambda i: (i, 0)),
        pl.BlockSpec(memory_space=pltpu.SMEM),
    ],
    out_shape=[
        jax.ShapeDtypeStruct((N_TOK, 1), jnp.int32),
        jax.ShapeDtypeStruct((1, 1), jnp.float32),
    ],
    scratch_shapes=[
        pltpu.SMEM((1, 1), jnp.float32),
    ],
)

CH = 128


@functools.cache
def _make_gather():
    info = plsc.get_sparse_core_info()
    nc = info.num_cores
    nw = info.num_cores * info.num_subcores
    bpw = N_TOK // nw
    nch = bpw // CH
    mesh = plsc.VectorSubcoreMesh(core_axis_name="c", subcore_axis_name="s")

    @functools.partial(
        pl.kernel,
        mesh=mesh,
        out_type=jax.ShapeDtypeStruct((N_TOK, D), jnp.float32),
        scratch_types=[
            pltpu.VMEM((nch, CH), jnp.int32),
            pltpu.VMEM((bpw, D), jnp.float32),
            pltpu.SemaphoreType.DMA,
        ],
    )
    def _gather(table_hbm, idx_hbm, out_hbm, idx_v, rows_v, sem):
        wid = lax.axis_index("s") * nc + lax.axis_index("c")
        pltpu.sync_copy(idx_hbm.at[pl.ds(wid * nch, nch)], idx_v)
        copies = []
        for c in range(nch):
            copies.append(pltpu.async_copy(
                table_hbm.at[idx_v.at[c]], rows_v.at[pl.ds(c * CH, CH)], sem))
        for cp in copies:
            cp.wait()
        pltpu.sync_copy(rows_v, out_hbm.at[pl.ds(wid * bpw, bpw)])

    return _gather


def kernel(inputs, embeddings):
    x = jnp.transpose(inputs, (0, 2, 3, 1))
    zs = x.reshape(-1, D) * (-2.0)
    z2 = 0.25 * jnp.sum(zs ** 2, axis=1, keepdims=True)
    e2 = jnp.sum(embeddings ** 2, axis=1, keepdims=True).T
    k, loss = _vq(zs, embeddings, z2, e2)
    q = _make_gather()(embeddings, k.reshape(N_TOK // CH, CH))
    output = q.reshape(8, 256, 32, 32)
    return (output, loss.reshape(()))


# speedup vs baseline: 1.5954x; 1.5954x over previous
import functools

import jax
import jax.numpy as jnp
from jax import lax
from jax.experimental import pallas as pl
from jax.experimental.pallas import tpu as pltpu
from jax.experimental.pallas import tpu_sc as plsc

N_EMB = 8192
D = 256
N_TOK = 8192
BT = 512
BC = 8192
GI = N_TOK // BT
GJ = N_EMB // BC
SCALE = 1.25 / (N_TOK * D)


def _vq_body(z_ref, en_ref, z2_ref, e2_ref, k_ref, loss_ref, acc_ref):
    i = pl.program_id(0)
    z = z_ref[...]
    z2 = z2_ref[...]
    lane = lax.broadcasted_iota(jnp.int32, (BT, 128), 1).astype(jnp.float32)
    rmin = None
    ridx = None
    for j in range(GJ):
        en = en_ref[pl.ds(j * BC, BC), :]
        dotj = lax.dot_general(z, en, (((1,), (1,)), ((), ())),
                               preferred_element_type=jnp.float32)
        scores = (z2 + dotj) + e2_ref[:, pl.ds(j * BC, BC)]
        m = scores[:, 0:128]
        c = jnp.zeros((BT, 128), jnp.float32)
        for t in range(1, BC // 128):
            st = scores[:, t * 128:(t + 1) * 128]
            lt = st < m
            m = jnp.where(lt, st, m)
            c = jnp.where(lt, jnp.float32(t), c)
        bmin = jnp.min(m, axis=1, keepdims=True)
        full = c * 128.0 + lane
        bidx = jnp.min(jnp.where(m == bmin, full, jnp.float32(BC)),
                       axis=1, keepdims=True).astype(jnp.int32) + j * BC
        if j == 0:
            rmin, ridx = bmin, bidx
        else:
            take = bmin < rmin
            rmin = jnp.where(take, bmin, rmin)
            ridx = jnp.where(take, bidx, ridx)
    k_ref[...] = ridx
    part = jnp.sum(rmin)
    tot = jnp.where(i == 0, part, acc_ref[0, 0] + part)
    acc_ref[0, 0] = tot

    @pl.when(i == GI - 1)
    def _():
        loss_ref[0, 0] = tot * SCALE


_vq = pl.pallas_call(
    _vq_body,
    grid=(GI,),
    in_specs=[
        pl.BlockSpec((BT, D), lambda i: (i, 0)),
        pl.BlockSpec((N_EMB, D), lambda i: (0, 0)),
        pl.BlockSpec((BT, 1), lambda i: (i, 0)),
        pl.BlockSpec((1, N_EMB), lambda i: (0, 0)),
    ],
    out_specs=[
        pl.BlockSpec((BT, 1), lambda i: (i, 0)),
        pl.BlockSpec(memory_space=pltpu.SMEM),
    ],
    out_shape=[
        jax.ShapeDtypeStruct((N_TOK, 1), jnp.int32),
        jax.ShapeDtypeStruct((1, 1), jnp.float32),
    ],
    scratch_shapes=[
        pltpu.SMEM((1, 1), jnp.float32),
    ],
)

CH = 128


@functools.cache
def _make_gather():
    info = plsc.get_sparse_core_info()
    nc = info.num_cores
    nw = info.num_cores * info.num_subcores
    bpw = N_TOK // nw
    nch = bpw // CH
    mesh = plsc.VectorSubcoreMesh(core_axis_name="c", subcore_axis_name="s")

    @functools.partial(
        pl.kernel,
        mesh=mesh,
        out_type=jax.ShapeDtypeStruct((N_TOK, D), jnp.float32),
        scratch_types=[
            pltpu.VMEM((nch, CH), jnp.int32),
            pltpu.VMEM((bpw, D), jnp.float32),
            pltpu.SemaphoreType.DMA,
        ],
    )
    def _gather(table_hbm, idx_hbm, out_hbm, idx_v, rows_v, sem):
        wid = lax.axis_index("s") * nc + lax.axis_index("c")
        pltpu.sync_copy(idx_hbm.at[pl.ds(wid * nch, nch)], idx_v)
        copies = []
        for c in range(nch):
            copies.append(pltpu.async_copy(
                table_hbm.at[idx_v.at[c]], rows_v.at[pl.ds(c * CH, CH)], sem))
        for cp in copies:
            cp.wait()
        pltpu.sync_copy(rows_v, out_hbm.at[pl.ds(wid * bpw, bpw)])

    return _gather


def kernel(inputs, embeddings):
    x = jnp.transpose(inputs, (0, 2, 3, 1))
    zs = x.reshape(-1, D) * (-2.0)
    z2 = 0.25 * jnp.sum(zs ** 2, axis=1, keepdims=True)
    e2 = jnp.sum(embeddings ** 2, axis=1, keepdims=True).T
    k, loss = _vq(zs, embeddings, z2, e2)
    q = _make_gather()(embeddings, k.reshape(N_TOK // CH, CH))
    qr = q.reshape(x.shape)
    output = jnp.transpose(qr, (0, 3, 1, 2))
    return (output, loss.reshape(()))
